# baseline (speedup 1.0000x reference)
import functools

import jax
import jax.numpy as jnp
from jax import lax
from jax.experimental import pallas as pl
from jax.experimental.pallas import tpu as pltpu
from jax.experimental.pallas import tpu_sc as plsc

NC, NS, LANES = 2, 16, 16
MSGW = 64
AW = 128
CHUNK = 64
EPR = 8


def _mm_bias_kernel(x_ref, w_ref, b_ref, o_ref):
    o_ref[...] = (
        jnp.dot(x_ref[...], w_ref[...], preferred_element_type=jnp.float32)
        + b_ref[...]
    )


def _final_kernel(node_ref, sums_ref, batch_ref, glob_ref, a1_ref, a2_ref,
                  a3_ref, ab_ref, o_ref):
    s = sums_ref[0] + sums_ref[1]
    cnt = s[:, MSGW:MSGW + 1]
    mean = s[:, :MSGW] / jnp.maximum(cnt, 1.0)
    g_tab = jnp.dot(glob_ref[...], a3_ref[...],
                    preferred_element_type=jnp.float32)
    b = batch_ref[...]
    nb = g_tab.shape[0]
    onehot = (b == lax.broadcasted_iota(jnp.int32, (b.shape[0], nb), 1))
    acc = (
        jnp.dot(node_ref[...], a1_ref[...], preferred_element_type=jnp.float32)
        + jnp.dot(mean, a2_ref[...], preferred_element_type=jnp.float32)
        + jnp.dot(onehot.astype(jnp.float32), g_tab,
                  preferred_element_type=jnp.float32)
        + ab_ref[...]
    )
    o_ref[...] = jnp.maximum(acc, 0.0)


@functools.lru_cache(maxsize=None)
def _build_sc_edges(N, E):
    W = NC * NS
    epw = E // W
    nch = epw // CHUNK
    qrows = CHUNK // EPR
    qw = EPR * MSGW
    rpt = (-(-N // NS) + 7) // 8 * 8
    npad = NS * rpt
    NB = 2

    mesh = plsc.VectorSubcoreMesh(core_axis_name="c", subcore_axis_name="s")

    @functools.partial(
        pl.kernel,
        out_type=jax.ShapeDtypeStruct((NC, npad, AW), jnp.float32),
        mesh=mesh,
        scratch_types=[
            pltpu.VMEM_SHARED((npad, AW), jnp.float32),
            pltpu.VMEM((epw,), jnp.int32),
            [pltpu.VMEM((CHUNK,), jnp.int32) for _ in range(NB)],
            [pltpu.VMEM((CHUNK, AW), jnp.float32) for _ in range(NB)],
            [pltpu.VMEM((qrows, qw), jnp.float32) for _ in range(NB)],
            [pltpu.SemaphoreType.DMA for _ in range(NB)],
            [pltpu.SemaphoreType.DMA for _ in range(NB)],
            [pltpu.SemaphoreType.DMA for _ in range(NB)],
            [pltpu.SemaphoreType.DMA for _ in range(NB)],
        ],
    )
    def sc_edges(p_hbm, q_hbm, src_hbm, dst_hbm, zero_hbm, out_hbm,
                 accum, sidx, dbuf, msg, qbuf, qsem, gsem, dsem, ssem):
        cid = lax.axis_index("c")
        sid = lax.axis_index("s")
        w = cid * NS + sid
        ebase = w * epw
        qrow0 = w * (epw // EPR)

        pltpu.sync_copy(src_hbm.at[pl.ds(ebase, epw)], sidx)
        pltpu.sync_copy(zero_hbm.at[pl.ds(sid * rpt, rpt)],
                        accum.at[pl.ds(sid * rpt, rpt)])

        plsc.subcore_barrier()

        def issue(jc, b):
            o = jc * CHUNK
            pltpu.async_copy(dst_hbm.at[pl.ds(ebase + o, CHUNK)],
                             dbuf[b], dsem[b])
            pltpu.async_copy(q_hbm.at[pl.ds(qrow0 + jc * qrows, qrows)],
                             qbuf[b], qsem[b])
            pltpu.async_copy(p_hbm.at[sidx.at[pl.ds(o, CHUNK)]],
                             msg[b], gsem[b])

        def wait_in(b):
            pltpu.make_async_copy(q_hbm.at[pl.ds(0, qrows)], qbuf[b],
                                  qsem[b]).wait()
            pltpu.make_async_copy(p_hbm.at[sidx.at[pl.ds(0, CHUNK)]], msg[b],
                                  gsem[b]).wait()
            pltpu.make_async_copy(dst_hbm.at[pl.ds(0, CHUNK)], dbuf[b],
                                  dsem[b]).wait()

        def process(jc, b):
            wait_in(b)

            @plsc.parallel_loop(0, qrows, unroll=2)
            def _relu(r):
                for e8 in range(EPR):
                    for jj in range(MSGW // LANES):
                        sl = pl.ds(jj * LANES, LANES)
                        sq = pl.ds(e8 * MSGW + jj * LANES, LANES)
                        i = r * EPR + e8
                        msg[b][i, sl] = jnp.maximum(
                            msg[b][i, sl] + qbuf[b][r, sq], 0.0)

            pltpu.async_copy(msg[b], accum.at[dbuf[b]], ssem[b], add=True)

        def wait_scatter(b):
            pltpu.make_async_copy(msg[b], accum.at[dbuf[b]], ssem[b]).wait()

        issue(0, 0)
        issue(1, 1)

        def round_body(k, carry):
            jc0 = 2 * k

            process(jc0, 0)

            @pl.when(jc0 + 2 < nch)
            def _():
                wait_scatter(0)
                issue(jc0 + 2, 0)

            process(jc0 + 1, 1)

            @pl.when(jc0 + 3 < nch)
            def _():
                wait_scatter(1)
                issue(jc0 + 3, 1)

            return carry

        lax.fori_loop(0, nch // 2, round_body, 0)
        if nch % 2:
            process(nch - 1, 0)
        wait_scatter(0)
        wait_scatter(1)

        plsc.subcore_barrier()

        pltpu.sync_copy(accum.at[pl.ds(sid * rpt, rpt)],
                        out_hbm.at[cid, pl.ds(sid * rpt, rpt)])

    return sc_edges


def kernel(node_feats, edge_index, edge_feats, glob_feats, batch, mW1, mb1,
           aW1, ab1):
    N, D = node_feats.shape
    E, EIN = edge_feats.shape
    B = glob_feats.shape[0]
    f32 = jnp.float32
    bf16 = jnp.bfloat16

    mW1a = mW1[:D]
    mW1b = mW1[D:]
    A1 = aW1[:D]
    A2 = aW1[D:D + MSGW]
    A3 = aW1[D + MSGW:]
    AOUT = aW1.shape[1]

    wpad = jnp.concatenate([mW1a, jnp.zeros((D, AW - MSGW), f32)], axis=1)
    bias_row = jnp.zeros((1, AW), f32).at[0, MSGW].set(1.0)
    rb = 2000
    P2 = pl.pallas_call(
        _mm_bias_kernel,
        grid=(N // rb,),
        in_specs=[
            pl.BlockSpec((rb, D), lambda i: (i, 0)),
            pl.BlockSpec((D, AW), lambda i: (0, 0)),
            pl.BlockSpec((1, AW), lambda i: (0, 0)),
        ],
        out_specs=pl.BlockSpec((rb, AW), lambda i: (i, 0)),
        out_shape=jax.ShapeDtypeStruct((N, AW), f32),
    )(node_feats, wpad, bias_row)

    W = NC * NS
    epw = -(-E // (W * CHUNK)) * CHUNK
    epad = W * epw
    rpt = (-(-N // NS) + 7) // 8 * 8
    npad = NS * rpt
    src_pad = jnp.concatenate(
        [edge_index[0], jnp.zeros((epad - E,), jnp.int32)])
    dst_pad = jnp.concatenate(
        [edge_index[1], jnp.full((epad - E,), N, jnp.int32)])

    efp = jnp.concatenate(
        [edge_feats, jnp.zeros((epad - E, EIN), edge_feats.dtype)])
    ef8 = efp.reshape(epad // EPR, EPR * EIN).astype(bf16)
    w8 = jnp.kron(jnp.eye(EPR, dtype=f32), mW1b).astype(bf16)
    b8 = jnp.tile(mb1, EPR).reshape(1, EPR * MSGW)
    nqrows = epad // EPR
    qb = nqrows
    for cand in range(2512, 15, -16):
        if nqrows % cand == 0:
            qb = cand
            break
    Q = pl.pallas_call(
        _mm_bias_kernel,
        grid=(nqrows // qb,),
        in_specs=[
            pl.BlockSpec((qb, EPR * EIN), lambda i: (i, 0)),
            pl.BlockSpec((EPR * EIN, EPR * MSGW), lambda i: (0, 0)),
            pl.BlockSpec((1, EPR * MSGW), lambda i: (0, 0)),
        ],
        out_specs=pl.BlockSpec((qb, EPR * MSGW), lambda i: (i, 0)),
        out_shape=jax.ShapeDtypeStruct((nqrows, EPR * MSGW), f32),
    )(ef8, w8, b8)

    zeros = jnp.zeros((npad, AW), f32)
    sums2 = _build_sc_edges(N, epad)(P2, Q, src_pad, dst_pad, zeros)
    sums2 = sums2[:, :N]

    fb = 2000
    out = pl.pallas_call(
        _final_kernel,
        grid=(N // fb,),
        in_specs=[
            pl.BlockSpec((fb, D), lambda i: (i, 0)),
            pl.BlockSpec((NC, fb, AW), lambda i: (0, i, 0)),
            pl.BlockSpec((fb, 1), lambda i: (i, 0)),
            pl.BlockSpec((B, EIN), lambda i: (0, 0)),
            pl.BlockSpec((D, AOUT), lambda i: (0, 0)),
            pl.BlockSpec((MSGW, AOUT), lambda i: (0, 0)),
            pl.BlockSpec((EIN, AOUT), lambda i: (0, 0)),
            pl.BlockSpec((1, AOUT), lambda i: (0, 0)),
        ],
        out_specs=pl.BlockSpec((fb, AOUT), lambda i: (i, 0)),
        out_shape=jax.ShapeDtypeStruct((N, AOUT), f32),
    )(node_feats, sums2, batch.reshape(N, 1), glob_feats, A1, A2, A3,
      ab1.reshape(1, AOUT))

    return out


# speedup vs baseline: 1.0655x; 1.0655x over previous
import functools

import jax
import jax.numpy as jnp
from jax import lax
from jax.experimental import pallas as pl
from jax.experimental.pallas import tpu as pltpu
from jax.experimental.pallas import tpu_sc as plsc

NC, NS, LANES = 2, 16, 16
MSGW = 64
AW = 128
CHUNK = 64
EPR = 8


def _mm_bias_kernel(x_ref, w_ref, b_ref, o_ref):
    o_ref[...] = (
        jnp.dot(x_ref[...], w_ref[...], preferred_element_type=jnp.float32)
        + b_ref[...]
    )


def _final_kernel(node_ref, sums_ref, batch_ref, glob_ref, a1_ref, a2_ref,
                  a3_ref, ab_ref, o_ref):
    s = sums_ref[0] + sums_ref[1]
    cnt = s[:, MSGW:MSGW + 1]
    mean = s[:, :MSGW] / jnp.maximum(cnt, 1.0)
    g_tab = jnp.dot(glob_ref[...], a3_ref[...],
                    preferred_element_type=jnp.float32)
    b = batch_ref[...]
    nb = g_tab.shape[0]
    onehot = (b == lax.broadcasted_iota(jnp.int32, (b.shape[0], nb), 1))
    acc = (
        jnp.dot(node_ref[...], a1_ref[...], preferred_element_type=jnp.float32)
        + jnp.dot(mean, a2_ref[...], preferred_element_type=jnp.float32)
        + jnp.dot(onehot.astype(jnp.float32), g_tab,
                  preferred_element_type=jnp.float32)
        + ab_ref[...]
    )
    o_ref[...] = jnp.maximum(acc, 0.0)


@functools.lru_cache(maxsize=None)
def _build_sc_edges(N, E):
    W = NC * NS
    epw = E // W
    nch = epw // CHUNK
    qrows = CHUNK // EPR
    qw = EPR * MSGW
    rpt = (-(-N // NS) + 7) // 8 * 8
    npad = NS * rpt
    NB = 3

    mesh = plsc.VectorSubcoreMesh(core_axis_name="c", subcore_axis_name="s")

    @functools.partial(
        pl.kernel,
        out_type=jax.ShapeDtypeStruct((NC, npad, AW), jnp.float32),
        mesh=mesh,
        scratch_types=[
            pltpu.VMEM_SHARED((npad, AW), jnp.float32),
            pltpu.VMEM((epw,), jnp.int32),
            [pltpu.VMEM((CHUNK,), jnp.int32) for _ in range(NB)],
            [pltpu.VMEM((CHUNK, AW), jnp.float32) for _ in range(NB)],
            [pltpu.VMEM((qrows, qw), jnp.float32) for _ in range(NB)],
            [pltpu.SemaphoreType.DMA for _ in range(NB)],
            [pltpu.SemaphoreType.DMA for _ in range(NB)],
            [pltpu.SemaphoreType.DMA for _ in range(NB)],
            [pltpu.SemaphoreType.DMA for _ in range(NB)],
        ],
    )
    def sc_edges(p_hbm, q_hbm, src_hbm, dst_hbm, zero_hbm, out_hbm,
                 accum, sidx, dbuf, msg, qbuf, qsem, gsem, dsem, ssem):
        cid = lax.axis_index("c")
        sid = lax.axis_index("s")
        w = cid * NS + sid
        ebase = w * epw
        qrow0 = w * (epw // EPR)

        pltpu.sync_copy(src_hbm.at[pl.ds(ebase, epw)], sidx)
        pltpu.sync_copy(zero_hbm.at[pl.ds(sid * rpt, rpt)],
                        accum.at[pl.ds(sid * rpt, rpt)])

        plsc.subcore_barrier()

        def issue(jc, b):
            o = jc * CHUNK
            pltpu.async_copy(dst_hbm.at[pl.ds(ebase + o, CHUNK)],
                             dbuf[b], dsem[b])
            pltpu.async_copy(q_hbm.at[pl.ds(qrow0 + jc * qrows, qrows)],
                             qbuf[b], qsem[b])
            pltpu.async_copy(p_hbm.at[sidx.at[pl.ds(o, CHUNK)]],
                             msg[b], gsem[b])

        def wait_in(b):
            pltpu.make_async_copy(q_hbm.at[pl.ds(0, qrows)], qbuf[b],
                                  qsem[b]).wait()
            pltpu.make_async_copy(p_hbm.at[sidx.at[pl.ds(0, CHUNK)]], msg[b],
                                  gsem[b]).wait()
            pltpu.make_async_copy(dst_hbm.at[pl.ds(0, CHUNK)], dbuf[b],
                                  dsem[b]).wait()

        def process(jc, b):
            wait_in(b)

            @plsc.parallel_loop(0, qrows, unroll=2)
            def _relu(r):
                for e8 in range(EPR):
                    for jj in range(MSGW // LANES):
                        sl = pl.ds(jj * LANES, LANES)
                        sq = pl.ds(e8 * MSGW + jj * LANES, LANES)
                        i = r * EPR + e8
                        msg[b][i, sl] = jnp.maximum(
                            msg[b][i, sl] + qbuf[b][r, sq], 0.0)

            pltpu.async_copy(msg[b], accum.at[dbuf[b]], ssem[b], add=True)

        def wait_scatter(b):
            pltpu.make_async_copy(msg[b], accum.at[dbuf[b]], ssem[b]).wait()

        for b in range(NB):
            issue(b, b)

        def round_body(k, carry):
            jc0 = NB * k
            for b in range(NB):
                process(jc0 + b, b)

                @pl.when(jc0 + b + NB < nch)
                def _(b=b):
                    wait_scatter(b)
                    issue(jc0 + b + NB, b)

            return carry

        lax.fori_loop(0, nch // NB, round_body, 0)
        for jc in range(nch - nch % NB, nch):
            process(jc, jc % NB)
        for b in range(NB):
            wait_scatter(b)

        plsc.subcore_barrier()

        pltpu.sync_copy(accum.at[pl.ds(sid * rpt, rpt)],
                        out_hbm.at[cid, pl.ds(sid * rpt, rpt)])

    return sc_edges


def kernel(node_feats, edge_index, edge_feats, glob_feats, batch, mW1, mb1,
           aW1, ab1):
    N, D = node_feats.shape
    E, EIN = edge_feats.shape
    B = glob_feats.shape[0]
    f32 = jnp.float32
    bf16 = jnp.bfloat16

    mW1a = mW1[:D]
    mW1b = mW1[D:]
    A1 = aW1[:D]
    A2 = aW1[D:D + MSGW]
    A3 = aW1[D + MSGW:]
    AOUT = aW1.shape[1]

    wpad = jnp.concatenate([mW1a, jnp.zeros((D, AW - MSGW), f32)], axis=1)
    bias_row = jnp.zeros((1, AW), f32).at[0, MSGW].set(1.0)
    rb = 2000
    P2 = pl.pallas_call(
        _mm_bias_kernel,
        grid=(N // rb,),
        in_specs=[
            pl.BlockSpec((rb, D), lambda i: (i, 0)),
            pl.BlockSpec((D, AW), lambda i: (0, 0)),
            pl.BlockSpec((1, AW), lambda i: (0, 0)),
        ],
        out_specs=pl.BlockSpec((rb, AW), lambda i: (i, 0)),
        out_shape=jax.ShapeDtypeStruct((N, AW), f32),
    )(node_feats, wpad, bias_row)

    W = NC * NS
    epw = -(-E // (W * CHUNK)) * CHUNK
    epad = W * epw
    rpt = (-(-N // NS) + 7) // 8 * 8
    npad = NS * rpt
    src_pad = jnp.concatenate(
        [edge_index[0], jnp.zeros((epad - E,), jnp.int32)])
    dst_pad = jnp.concatenate(
        [edge_index[1], N + jnp.arange(epad - E, dtype=jnp.int32)
         % max(npad - N, 1)])

    ef8 = jnp.pad(
        edge_feats.reshape(E // EPR, EPR * EIN),
        ((0, (epad - E) // EPR), (0, 0))).astype(bf16)
    w8 = jnp.kron(jnp.eye(EPR, dtype=f32), mW1b).astype(bf16)
    b8 = jnp.tile(mb1, EPR).reshape(1, EPR * MSGW)
    nqrows = epad // EPR
    qb = nqrows
    for cand in range(2512, 15, -16):
        if nqrows % cand == 0:
            qb = cand
            break
    Q = pl.pallas_call(
        _mm_bias_kernel,
        grid=(nqrows // qb,),
        in_specs=[
            pl.BlockSpec((qb, EPR * EIN), lambda i: (i, 0)),
            pl.BlockSpec((EPR * EIN, EPR * MSGW), lambda i: (0, 0)),
            pl.BlockSpec((1, EPR * MSGW), lambda i: (0, 0)),
        ],
        out_specs=pl.BlockSpec((qb, EPR * MSGW), lambda i: (i, 0)),
        out_shape=jax.ShapeDtypeStruct((nqrows, EPR * MSGW), f32),
    )(ef8, w8, b8)

    zeros = jnp.zeros((npad, AW), f32)
    sums2 = _build_sc_edges(N, epad)(P2, Q, src_pad, dst_pad, zeros)
    sums2 = sums2[:, :N]

    fb = 2000
    out = pl.pallas_call(
        _final_kernel,
        grid=(N // fb,),
        in_specs=[
            pl.BlockSpec((fb, D), lambda i: (i, 0)),
            pl.BlockSpec((NC, fb, AW), lambda i: (0, i, 0)),
            pl.BlockSpec((fb, 1), lambda i: (i, 0)),
            pl.BlockSpec((B, EIN), lambda i: (0, 0)),
            pl.BlockSpec((D, AOUT), lambda i: (0, 0)),
            pl.BlockSpec((MSGW, AOUT), lambda i: (0, 0)),
            pl.BlockSpec((EIN, AOUT), lambda i: (0, 0)),
            pl.BlockSpec((1, AOUT), lambda i: (0, 0)),
        ],
        out_specs=pl.BlockSpec((fb, AOUT), lambda i: (i, 0)),
        out_shape=jax.ShapeDtypeStruct((N, AOUT), f32),
    )(node_feats, sums2, batch.reshape(N, 1), glob_feats, A1, A2, A3,
      ab1.reshape(1, AOUT))

    return out


# speedup vs baseline: 1.0853x; 1.0186x over previous
import functools

import jax
import jax.numpy as jnp
from jax import lax
from jax.experimental import pallas as pl
from jax.experimental.pallas import tpu as pltpu
from jax.experimental.pallas import tpu_sc as plsc

NC, NS, LANES = 2, 16, 16
MSGW = 64
AW = 128
CHUNK = 64
EPR = 8


def _mm_bias_kernel(x_ref, w_ref, b_ref, o_ref):
    o_ref[...] = (
        jnp.dot(x_ref[...], w_ref[...], preferred_element_type=jnp.float32)
        + b_ref[...]
    )


def _final_kernel(node_ref, sums_ref, batch_ref, glob_ref, a1_ref, a2_ref,
                  a3_ref, ab_ref, o_ref):
    s = sums_ref[0] + sums_ref[1]
    cnt = s[:, MSGW:MSGW + 1]
    mean = s[:, :MSGW] / jnp.maximum(cnt, 1.0)
    g_tab = jnp.dot(glob_ref[...], a3_ref[...],
                    preferred_element_type=jnp.float32)
    b = batch_ref[...]
    nb = g_tab.shape[0]
    onehot = (b == lax.broadcasted_iota(jnp.int32, (b.shape[0], nb), 1))
    acc = (
        jnp.dot(node_ref[...], a1_ref[...], preferred_element_type=jnp.float32)
        + jnp.dot(mean, a2_ref[...], preferred_element_type=jnp.float32)
        + jnp.dot(onehot.astype(jnp.float32), g_tab,
                  preferred_element_type=jnp.float32)
        + ab_ref[...]
    )
    o_ref[...] = jnp.maximum(acc, 0.0)


@functools.lru_cache(maxsize=None)
def _build_sc_edges(N, E):
    W = NC * NS
    epw = E // W
    nch = epw // CHUNK
    qrows = CHUNK // EPR
    qw = EPR * MSGW
    rpt = (-(-N // NS) + 7) // 8 * 8
    npad = NS * rpt
    NB = 2

    mesh = plsc.VectorSubcoreMesh(core_axis_name="c", subcore_axis_name="s")

    @functools.partial(
        pl.kernel,
        out_type=jax.ShapeDtypeStruct((NC, npad, AW), jnp.float32),
        mesh=mesh,
        scratch_types=[
            pltpu.VMEM_SHARED((npad, AW), jnp.float32),
            pltpu.VMEM((epw,), jnp.int32),
            [pltpu.VMEM((CHUNK,), jnp.int32) for _ in range(NB)],
            [pltpu.VMEM((CHUNK, AW), jnp.float32) for _ in range(NB)],
            [pltpu.VMEM((qrows, qw), jnp.float32) for _ in range(NB)],
            [pltpu.SemaphoreType.DMA for _ in range(NB)],
            [pltpu.SemaphoreType.DMA for _ in range(NB)],
            [pltpu.SemaphoreType.DMA for _ in range(NB)],
            [pltpu.SemaphoreType.DMA for _ in range(NB)],
        ],
    )
    def sc_edges(p_hbm, q_hbm, src_hbm, dst_hbm, zero_hbm, out_hbm,
                 accum, sidx, dbuf, msg, qbuf, qsem, gsem, dsem, ssem):
        cid = lax.axis_index("c")
        sid = lax.axis_index("s")
        w = cid * NS + sid
        ebase = w * epw
        qrow0 = w * (epw // EPR)

        pltpu.sync_copy(src_hbm.at[pl.ds(ebase, epw)], sidx)
        pltpu.sync_copy(zero_hbm.at[pl.ds(sid * rpt, rpt)],
                        accum.at[pl.ds(sid * rpt, rpt)])

        plsc.subcore_barrier()

        def issue(jc, b):
            o = jc * CHUNK
            pltpu.async_copy(dst_hbm.at[pl.ds(ebase + o, CHUNK)],
                             dbuf[b], dsem[b])
            pltpu.async_copy(q_hbm.at[pl.ds(qrow0 + jc * qrows, qrows)],
                             qbuf[b], qsem[b])
            pltpu.async_copy(p_hbm.at[sidx.at[pl.ds(o, CHUNK)]],
                             msg[b], gsem[b])

        def wait_in(b):
            pltpu.make_async_copy(q_hbm.at[pl.ds(0, qrows)], qbuf[b],
                                  qsem[b]).wait()
            pltpu.make_async_copy(p_hbm.at[sidx.at[pl.ds(0, CHUNK)]], msg[b],
                                  gsem[b]).wait()
            pltpu.make_async_copy(dst_hbm.at[pl.ds(0, CHUNK)], dbuf[b],
                                  dsem[b]).wait()

        def process(jc, b):
            wait_in(b)

            @plsc.parallel_loop(0, qrows, unroll=2)
            def _relu(r):
                for e8 in range(EPR):
                    for jj in range(MSGW // LANES):
                        sl = pl.ds(jj * LANES, LANES)
                        sq = pl.ds(e8 * MSGW + jj * LANES, LANES)
                        i = r * EPR + e8
                        msg[b][i, sl] = jnp.maximum(
                            msg[b][i, sl] + qbuf[b][r, sq], 0.0)

            pltpu.async_copy(msg[b], accum.at[dbuf[b]], ssem[b], add=True)

        def wait_scatter(b):
            pltpu.make_async_copy(msg[b], accum.at[dbuf[b]], ssem[b]).wait()

        for b in range(NB):
            issue(b, b)

        def round_body(k, carry):
            jc0 = NB * k
            for b in range(NB):
                process(jc0 + b, b)

                @pl.when(jc0 + b + NB < nch)
                def _(b=b):
                    wait_scatter(b)
                    issue(jc0 + b + NB, b)

            return carry

        lax.fori_loop(0, nch // NB, round_body, 0)
        for jc in range(nch - nch % NB, nch):
            process(jc, jc % NB)
        for b in range(NB):
            wait_scatter(b)

        plsc.subcore_barrier()

        pltpu.sync_copy(accum.at[pl.ds(sid * rpt, rpt)],
                        out_hbm.at[cid, pl.ds(sid * rpt, rpt)])

    return sc_edges


def kernel(node_feats, edge_index, edge_feats, glob_feats, batch, mW1, mb1,
           aW1, ab1):
    N, D = node_feats.shape
    E, EIN = edge_feats.shape
    B = glob_feats.shape[0]
    f32 = jnp.float32
    bf16 = jnp.bfloat16

    mW1a = mW1[:D]
    mW1b = mW1[D:]
    A1 = aW1[:D]
    A2 = aW1[D:D + MSGW]
    A3 = aW1[D + MSGW:]
    AOUT = aW1.shape[1]

    wpad = jnp.concatenate([mW1a, jnp.zeros((D, AW - MSGW), f32)], axis=1)
    bias_row = jnp.zeros((1, AW), f32).at[0, MSGW].set(1.0)
    rb = 2000
    P2 = pl.pallas_call(
        _mm_bias_kernel,
        grid=(N // rb,),
        in_specs=[
            pl.BlockSpec((rb, D), lambda i: (i, 0)),
            pl.BlockSpec((D, AW), lambda i: (0, 0)),
            pl.BlockSpec((1, AW), lambda i: (0, 0)),
        ],
        out_specs=pl.BlockSpec((rb, AW), lambda i: (i, 0)),
        out_shape=jax.ShapeDtypeStruct((N, AW), f32),
    )(node_feats, wpad, bias_row)

    W = NC * NS
    epw = -(-E // (W * CHUNK)) * CHUNK
    epad = W * epw
    rpt = (-(-N // NS) + 7) // 8 * 8
    npad = NS * rpt
    src_pad = jnp.concatenate(
        [edge_index[0], jnp.zeros((epad - E,), jnp.int32)])
    dst_pad = jnp.concatenate(
        [edge_index[1], N + jnp.arange(epad - E, dtype=jnp.int32)
         % max(npad - N, 1)])

    ef8 = jnp.pad(
        edge_feats.reshape(E // EPR, EPR * EIN),
        ((0, (epad - E) // EPR), (0, 0))).astype(bf16)
    w8 = jnp.kron(jnp.eye(EPR, dtype=f32), mW1b).astype(bf16)
    b8 = jnp.tile(mb1, EPR).reshape(1, EPR * MSGW)
    nqrows = epad // EPR
    qb = nqrows
    for cand in range(2512, 15, -16):
        if nqrows % cand == 0:
            qb = cand
            break
    Q = pl.pallas_call(
        _mm_bias_kernel,
        grid=(nqrows // qb,),
        in_specs=[
            pl.BlockSpec((qb, EPR * EIN), lambda i: (i, 0)),
            pl.BlockSpec((EPR * EIN, EPR * MSGW), lambda i: (0, 0)),
            pl.BlockSpec((1, EPR * MSGW), lambda i: (0, 0)),
        ],
        out_specs=pl.BlockSpec((qb, EPR * MSGW), lambda i: (i, 0)),
        out_shape=jax.ShapeDtypeStruct((nqrows, EPR * MSGW), f32),
    )(ef8, w8, b8)

    zeros = jnp.zeros((npad, AW), f32)
    sums2 = _build_sc_edges(N, epad)(P2, Q, src_pad, dst_pad, zeros)
    sums2 = sums2[:, :N]

    fb = 2000
    out = pl.pallas_call(
        _final_kernel,
        grid=(N // fb,),
        in_specs=[
            pl.BlockSpec((fb, D), lambda i: (i, 0)),
            pl.BlockSpec((NC, fb, AW), lambda i: (0, i, 0)),
            pl.BlockSpec((fb, 1), lambda i: (i, 0)),
            pl.BlockSpec((B, EIN), lambda i: (0, 0)),
            pl.BlockSpec((D, AOUT), lambda i: (0, 0)),
            pl.BlockSpec((MSGW, AOUT), lambda i: (0, 0)),
            pl.BlockSpec((EIN, AOUT), lambda i: (0, 0)),
            pl.BlockSpec((1, AOUT), lambda i: (0, 0)),
        ],
        out_specs=pl.BlockSpec((fb, AOUT), lambda i: (i, 0)),
        out_shape=jax.ShapeDtypeStruct((N, AOUT), f32),
    )(node_feats, sums2, batch.reshape(N, 1), glob_feats, A1, A2, A3,
      ab1.reshape(1, AOUT))

    return out


# speedup vs baseline: 1.0883x; 1.0028x over previous
import functools

import jax
import jax.numpy as jnp
from jax import lax
from jax.experimental import pallas as pl
from jax.experimental.pallas import tpu as pltpu
from jax.experimental.pallas import tpu_sc as plsc

NC, NS, LANES = 2, 16, 16
MSGW = 64
AW = 128
CHUNK = 64
EPR = 8


def _mm_bias_kernel(x_ref, w_ref, b_ref, o_ref):
    o_ref[...] = (
        jnp.dot(x_ref[...], w_ref[...], preferred_element_type=jnp.float32)
        + b_ref[...]
    )


def _final_kernel(node_ref, sums_ref, batch_ref, glob_ref, a1_ref, a2_ref,
                  a3_ref, ab_ref, o_ref):
    s = sums_ref[0] + sums_ref[1]
    cnt = s[:, MSGW:MSGW + 1]
    mean = s[:, :MSGW] / jnp.maximum(cnt, 1.0)
    g_tab = jnp.dot(glob_ref[...], a3_ref[...],
                    preferred_element_type=jnp.float32)
    b = batch_ref[...]
    nb = g_tab.shape[0]
    onehot = (b == lax.broadcasted_iota(jnp.int32, (b.shape[0], nb), 1))
    acc = (
        jnp.dot(node_ref[...], a1_ref[...], preferred_element_type=jnp.float32)
        + jnp.dot(mean, a2_ref[...], preferred_element_type=jnp.float32)
        + jnp.dot(onehot.astype(jnp.float32), g_tab,
                  preferred_element_type=jnp.float32)
        + ab_ref[...]
    )
    o_ref[...] = jnp.maximum(acc, 0.0)


@functools.lru_cache(maxsize=None)
def _build_sc_edges(N, E):
    W = NC * NS
    epw = E // W
    nch = epw // CHUNK
    qrows = CHUNK // EPR
    qw = EPR * MSGW
    rpt = (-(-N // NS) + 7) // 8 * 8
    npad = NS * rpt
    NB = 2

    mesh = plsc.VectorSubcoreMesh(core_axis_name="c", subcore_axis_name="s")

    @functools.partial(
        pl.kernel,
        out_type=jax.ShapeDtypeStruct((NC, npad, AW), jnp.float32),
        mesh=mesh,
        scratch_types=[
            pltpu.VMEM_SHARED((npad, AW), jnp.float32),
            pltpu.VMEM((epw,), jnp.int32),
            [pltpu.VMEM((CHUNK // 2,), jnp.int32) for _ in range(NB)],
            [pltpu.VMEM((CHUNK // 2,), jnp.int32) for _ in range(NB)],
            [pltpu.VMEM((CHUNK, AW), jnp.float32) for _ in range(NB)],
            [pltpu.VMEM((qrows, qw), jnp.float32) for _ in range(NB)],
            [pltpu.SemaphoreType.DMA for _ in range(NB)],
            [pltpu.SemaphoreType.DMA for _ in range(NB)],
            [pltpu.SemaphoreType.DMA for _ in range(NB)],
            [pltpu.SemaphoreType.DMA for _ in range(NB)],
            [pltpu.SemaphoreType.DMA for _ in range(NB)],
            [pltpu.SemaphoreType.DMA for _ in range(NB)],
            [pltpu.SemaphoreType.DMA for _ in range(NB)],
        ],
    )
    def sc_edges(p_hbm, q_hbm, src_hbm, dst_hbm, zero_hbm, out_hbm,
                 accum, sidx, dbufl, dbufh, msg, qbuf, qsem, gsem, gsem2,
                 dsem, dsem2, ssem, ssem2):
        cid = lax.axis_index("c")
        sid = lax.axis_index("s")
        w = cid * NS + sid
        ebase = w * epw
        qrow0 = w * (epw // EPR)

        pltpu.sync_copy(src_hbm.at[pl.ds(ebase, epw)], sidx)
        pltpu.sync_copy(zero_hbm.at[pl.ds(sid * rpt, rpt)],
                        accum.at[pl.ds(sid * rpt, rpt)])

        plsc.subcore_barrier()

        H = CHUNK // 2

        def issue(jc, b):
            o = jc * CHUNK
            pltpu.async_copy(dst_hbm.at[pl.ds(ebase + o, H)],
                             dbufl[b], dsem[b])
            pltpu.async_copy(dst_hbm.at[pl.ds(ebase + o + H, H)],
                             dbufh[b], dsem2[b])
            pltpu.async_copy(q_hbm.at[pl.ds(qrow0 + jc * qrows, qrows)],
                             qbuf[b], qsem[b])
            pltpu.async_copy(p_hbm.at[sidx.at[pl.ds(o, H)]],
                             msg[b].at[pl.ds(0, H)], gsem[b])
            pltpu.async_copy(p_hbm.at[sidx.at[pl.ds(o + H, H)]],
                             msg[b].at[pl.ds(H, H)], gsem2[b])

        def wait_in(b):
            pltpu.make_async_copy(q_hbm.at[pl.ds(0, qrows)], qbuf[b],
                                  qsem[b]).wait()
            pltpu.make_async_copy(p_hbm.at[sidx.at[pl.ds(0, H)]],
                                  msg[b].at[pl.ds(0, H)], gsem[b]).wait()
            pltpu.make_async_copy(p_hbm.at[sidx.at[pl.ds(0, H)]],
                                  msg[b].at[pl.ds(H, H)], gsem2[b]).wait()
            pltpu.make_async_copy(dst_hbm.at[pl.ds(0, H)], dbufl[b],
                                  dsem[b]).wait()
            pltpu.make_async_copy(dst_hbm.at[pl.ds(0, H)], dbufh[b],
                                  dsem2[b]).wait()

        def process(jc, b):
            wait_in(b)

            @plsc.parallel_loop(0, qrows, unroll=2)
            def _relu(r):
                for e8 in range(EPR):
                    for jj in range(MSGW // LANES):
                        sl = pl.ds(jj * LANES, LANES)
                        sq = pl.ds(e8 * MSGW + jj * LANES, LANES)
                        i = r * EPR + e8
                        msg[b][i, sl] = jnp.maximum(
                            msg[b][i, sl] + qbuf[b][r, sq], 0.0)

            pltpu.async_copy(msg[b].at[pl.ds(0, H)],
                             accum.at[dbufl[b]], ssem[b], add=True)
            pltpu.async_copy(msg[b].at[pl.ds(H, H)],
                             accum.at[dbufh[b]], ssem2[b], add=True)

        def wait_scatter(b):
            pltpu.make_async_copy(msg[b].at[pl.ds(0, H)],
                                  accum.at[dbufl[b]], ssem[b]).wait()
            pltpu.make_async_copy(msg[b].at[pl.ds(H, H)],
                                  accum.at[dbufh[b]], ssem2[b]).wait()

        for b in range(NB):
            issue(b, b)

        def round_body(k, carry):
            jc0 = NB * k
            for b in range(NB):
                process(jc0 + b, b)

                @pl.when(jc0 + b + NB < nch)
                def _(b=b):
                    wait_scatter(b)
                    issue(jc0 + b + NB, b)

            return carry

        lax.fori_loop(0, nch // NB, round_body, 0)
        for jc in range(nch - nch % NB, nch):
            process(jc, jc % NB)
        for b in range(NB):
            wait_scatter(b)

        plsc.subcore_barrier()

        pltpu.sync_copy(accum.at[pl.ds(sid * rpt, rpt)],
                        out_hbm.at[cid, pl.ds(sid * rpt, rpt)])

    return sc_edges


def kernel(node_feats, edge_index, edge_feats, glob_feats, batch, mW1, mb1,
           aW1, ab1):
    N, D = node_feats.shape
    E, EIN = edge_feats.shape
    B = glob_feats.shape[0]
    f32 = jnp.float32
    bf16 = jnp.bfloat16

    mW1a = mW1[:D]
    mW1b = mW1[D:]
    A1 = aW1[:D]
    A2 = aW1[D:D + MSGW]
    A3 = aW1[D + MSGW:]
    AOUT = aW1.shape[1]

    wpad = jnp.concatenate([mW1a, jnp.zeros((D, AW - MSGW), f32)], axis=1)
    bias_row = jnp.zeros((1, AW), f32).at[0, MSGW].set(1.0)
    rb = 2000
    P2 = pl.pallas_call(
        _mm_bias_kernel,
        grid=(N // rb,),
        in_specs=[
            pl.BlockSpec((rb, D), lambda i: (i, 0)),
            pl.BlockSpec((D, AW), lambda i: (0, 0)),
            pl.BlockSpec((1, AW), lambda i: (0, 0)),
        ],
        out_specs=pl.BlockSpec((rb, AW), lambda i: (i, 0)),
        out_shape=jax.ShapeDtypeStruct((N, AW), f32),
    )(node_feats, wpad, bias_row)

    W = NC * NS
    epw = -(-E // (W * CHUNK)) * CHUNK
    epad = W * epw
    rpt = (-(-N // NS) + 7) // 8 * 8
    npad = NS * rpt
    src_pad = jnp.concatenate(
        [edge_index[0], jnp.zeros((epad - E,), jnp.int32)])
    dst_pad = jnp.concatenate(
        [edge_index[1], N + jnp.arange(epad - E, dtype=jnp.int32)
         % max(npad - N, 1)])

    ef8 = jnp.pad(
        edge_feats.reshape(E // EPR, EPR * EIN),
        ((0, (epad - E) // EPR), (0, 0))).astype(bf16)
    w8 = jnp.kron(jnp.eye(EPR, dtype=f32), mW1b).astype(bf16)
    b8 = jnp.tile(mb1, EPR).reshape(1, EPR * MSGW)
    nqrows = epad // EPR
    qb = nqrows
    for cand in range(2512, 15, -16):
        if nqrows % cand == 0:
            qb = cand
            break
    Q = pl.pallas_call(
        _mm_bias_kernel,
        grid=(nqrows // qb,),
        in_specs=[
            pl.BlockSpec((qb, EPR * EIN), lambda i: (i, 0)),
            pl.BlockSpec((EPR * EIN, EPR * MSGW), lambda i: (0, 0)),
            pl.BlockSpec((1, EPR * MSGW), lambda i: (0, 0)),
        ],
        out_specs=pl.BlockSpec((qb, EPR * MSGW), lambda i: (i, 0)),
        out_shape=jax.ShapeDtypeStruct((nqrows, EPR * MSGW), f32),
    )(ef8, w8, b8)

    zeros = jnp.zeros((npad, AW), f32)
    sums2 = _build_sc_edges(N, epad)(P2, Q, src_pad, dst_pad, zeros)
    sums2 = sums2[:, :N]

    fb = 2000
    out = pl.pallas_call(
        _final_kernel,
        grid=(N // fb,),
        in_specs=[
            pl.BlockSpec((fb, D), lambda i: (i, 0)),
            pl.BlockSpec((NC, fb, AW), lambda i: (0, i, 0)),
            pl.BlockSpec((fb, 1), lambda i: (i, 0)),
            pl.BlockSpec((B, EIN), lambda i: (0, 0)),
            pl.BlockSpec((D, AOUT), lambda i: (0, 0)),
            pl.BlockSpec((MSGW, AOUT), lambda i: (0, 0)),
            pl.BlockSpec((EIN, AOUT), lambda i: (0, 0)),
            pl.BlockSpec((1, AOUT), lambda i: (0, 0)),
        ],
        out_specs=pl.BlockSpec((fb, AOUT), lambda i: (i, 0)),
        out_shape=jax.ShapeDtypeStruct((N, AOUT), f32),
    )(node_feats, sums2, batch.reshape(N, 1), glob_feats, A1, A2, A3,
      ab1.reshape(1, AOUT))

    return out
